# baseline (speedup 1.0000x reference)
import functools

import jax
import jax.numpy as jnp
from jax import lax
from jax.experimental import pallas as pl
from jax.experimental.pallas import tpu as pltpu
from jax.experimental.pallas import tpu_sc as plsc

BATCH = 4096
SEQ = 200
EMBED_DIM = 128
LANES = 16
NCHUNK = EMBED_DIM // LANES

NUM_CORES = 2
NUM_SUBCORES = 16
NW = NUM_CORES * NUM_SUBCORES
ROWS_PER_W = BATCH // NW
IDX_SPLIT = 2
IDX_CHUNK = SEQ // IDX_SPLIT
NBUF = 4
GROUP = 32
NGROUP = ROWS_PER_W // GROUP


def _body(x_hbm, table_hbm, out_hbm, idx_v, buf_v, out_v,
          g0, g1, g2, g3, i0, i1, o0, o1):
    wid = lax.axis_index("s") * NUM_CORES + lax.axis_index("c")
    base = wid * ROWS_PER_W
    gsems = (g0, g1, g2, g3)
    isems = (i0, i1)
    osems = (o0, o1)

    inv_n = jnp.float32(1.0 / SEQ)

    def issue_idx(gp):
        pltpu.async_copy(
            x_hbm.at[pl.ds(base + gp * GROUP, GROUP)],
            idx_v.at[gp % 2],
            isems[gp % 2],
        )

    def wait_idx(gp):
        pltpu.make_async_copy(
            x_hbm.at[pl.ds(base, GROUP)], idx_v.at[gp % 2], isems[gp % 2]
        ).wait()

    def issue_out(gp):
        pltpu.async_copy(
            out_v.at[gp % 2],
            out_hbm.at[pl.ds(base + gp * GROUP, GROUP)],
            osems[gp % 2],
        )

    def wait_out(gp):
        pltpu.make_async_copy(
            out_v.at[gp % 2], out_hbm.at[pl.ds(base, GROUP)], osems[gp % 2]
        ).wait()

    def gather_row(r, islot, slot):
        for c in range(IDX_SPLIT):
            pltpu.async_copy(
                table_hbm.at[idx_v.at[islot, r % GROUP, c]],
                buf_v.at[slot, c],
                gsems[slot],
            )

    def wait_row(slot):
        for c in range(IDX_SPLIT):
            pltpu.make_async_copy(
                table_hbm.at[idx_v.at[0, 0, c]], buf_v.at[slot, c], gsems[slot]
            ).wait()

    def accum_row(r, slot, oslot):
        def accum(j, carry):
            return tuple(
                carry[c * NCHUNK + d]
                + buf_v[slot, c, j, pl.ds(d * LANES, LANES)]
                for c in range(IDX_SPLIT)
                for d in range(NCHUNK)
            )

        init = tuple(
            jnp.zeros((LANES,), jnp.float32) for _ in range(IDX_SPLIT * NCHUNK)
        )
        acc = lax.fori_loop(0, IDX_CHUNK, accum, init, unroll=4)
        for d in range(NCHUNK):
            out_v[oslot, r % GROUP, pl.ds(d * LANES, LANES)] = (
                acc[d] + acc[NCHUNK + d]
            ) * inv_n

    DIST = NBUF - 1

    issue_idx(0)
    wait_idx(0)
    issue_idx(1)
    for r in range(DIST):
        gather_row(r, 0, r)

    for gp in range(NGROUP):
        gbase = gp * GROUP
        oslot = gp % 2
        if gp >= 2:
            wait_out(gp)
        if 2 <= gp + 1 <= NGROUP - 1:
            issue_idx(gp + 1)

        @pl.loop(gbase, gbase + GROUP - NBUF, step=NBUF)
        def main(r0):
            for b in range(NBUF):
                r = r0 + b
                gather_row(r + DIST, gp % 2, (b + DIST) % NBUF)
                wait_row(b)
                accum_row(r, b, oslot)

        for b in range(NBUF):
            r = gbase + GROUP - NBUF + b
            t = r + DIST
            if t < gbase + GROUP:
                gather_row(t, gp % 2, (b + DIST) % NBUF)
            elif gp + 1 < NGROUP:
                if t == gbase + GROUP:
                    wait_idx(gp + 1)
                gather_row(t, (gp + 1) % 2, (b + DIST) % NBUF)
            wait_row(b)
            accum_row(r, b, oslot)
        issue_out(gp)

    wait_out(NGROUP - 2)
    wait_out(NGROUP - 1)


@jax.jit
def kernel(x, table):
    x3 = x.astype(jnp.int32).reshape(BATCH, IDX_SPLIT, IDX_CHUNK)
    mesh = plsc.VectorSubcoreMesh(core_axis_name="c", subcore_axis_name="s")
    k = functools.partial(
        pl.kernel,
        out_type=jax.ShapeDtypeStruct((BATCH, EMBED_DIM), jnp.float32),
        mesh=mesh,
        scratch_types=[
            pltpu.VMEM((2, GROUP, IDX_SPLIT, IDX_CHUNK), jnp.int32),
            pltpu.VMEM((NBUF, IDX_SPLIT, IDX_CHUNK, EMBED_DIM), jnp.float32),
            pltpu.VMEM((2, GROUP, EMBED_DIM), jnp.float32),
            pltpu.SemaphoreType.DMA,
            pltpu.SemaphoreType.DMA,
            pltpu.SemaphoreType.DMA,
            pltpu.SemaphoreType.DMA,
            pltpu.SemaphoreType.DMA,
            pltpu.SemaphoreType.DMA,
            pltpu.SemaphoreType.DMA,
            pltpu.SemaphoreType.DMA,
        ],
    )(_body)
    return k(x3, table)


# speedup vs baseline: 1.0374x; 1.0374x over previous
import functools

import jax
import jax.numpy as jnp
from jax import lax
from jax.experimental import pallas as pl
from jax.experimental.pallas import tpu as pltpu
from jax.experimental.pallas import tpu_sc as plsc

BATCH = 4096
SEQ = 200
EMBED_DIM = 128
LANES = 16
NCHUNK = EMBED_DIM // LANES

NUM_CORES = 2
NUM_SUBCORES = 16
NW = NUM_CORES * NUM_SUBCORES
ROWS_PER_W = BATCH // NW
IDX_SPLIT = 2
IDX_CHUNK = SEQ // IDX_SPLIT


NBUF = 3


def _body(x_hbm, table_hbm, out_hbm, idx_v, buf_v, out_v, sem0, sem1, sem2):
    wid = lax.axis_index("s") * NUM_CORES + lax.axis_index("c")
    base = wid * ROWS_PER_W
    sems = (sem0, sem1, sem2)

    pltpu.sync_copy(x_hbm.at[pl.ds(base, ROWS_PER_W)], idx_v)

    inv_n = jnp.float32(1.0 / SEQ)

    def gather_row(r, slot):
        for c in range(IDX_SPLIT):
            pltpu.async_copy(
                table_hbm.at[idx_v.at[r, c]], buf_v.at[slot, c], sems[slot]
            )

    def wait_row(slot):
        for c in range(IDX_SPLIT):
            pltpu.make_async_copy(
                table_hbm.at[idx_v.at[0, c]], buf_v.at[slot, c], sems[slot]
            ).wait()

    def accum_row(r, slot):
        def accum(j, carry):
            return tuple(
                carry[c * NCHUNK + d]
                + buf_v[slot, c, j, pl.ds(d * LANES, LANES)]
                for c in range(IDX_SPLIT)
                for d in range(NCHUNK)
            )

        init = tuple(
            jnp.zeros((LANES,), jnp.float32) for _ in range(IDX_SPLIT * NCHUNK)
        )
        acc = lax.fori_loop(0, IDX_CHUNK, accum, init, unroll=4)
        for d in range(NCHUNK):
            out_v[r, pl.ds(d * LANES, LANES)] = (acc[d] + acc[NCHUNK + d]) * inv_n

    gather_row(0, 0)
    gather_row(1, 1)

    main_rows = ROWS_PER_W - (ROWS_PER_W % NBUF)

    @pl.loop(0, main_rows, step=NBUF)
    def ring(r0):
        for b in range(NBUF):
            r = r0 + b

            @pl.when(r + 2 < ROWS_PER_W)
            def _prefetch():
                gather_row(r + 2, (b + 2) % NBUF)

            wait_row(b)
            accum_row(r, b)

    for r in range(main_rows, ROWS_PER_W):
        wait_row(r % NBUF)
        accum_row(r, r % NBUF)

    pltpu.sync_copy(out_v, out_hbm.at[pl.ds(base, ROWS_PER_W)])


@jax.jit
def kernel(x, table):
    x3 = x.astype(jnp.int32).reshape(BATCH, IDX_SPLIT, IDX_CHUNK)
    mesh = plsc.VectorSubcoreMesh(core_axis_name="c", subcore_axis_name="s")
    k = functools.partial(
        pl.kernel,
        out_type=jax.ShapeDtypeStruct((BATCH, EMBED_DIM), jnp.float32),
        mesh=mesh,
        scratch_types=[
            pltpu.VMEM((ROWS_PER_W, IDX_SPLIT, IDX_CHUNK), jnp.int32),
            pltpu.VMEM((NBUF, IDX_SPLIT, IDX_CHUNK, EMBED_DIM), jnp.float32),
            pltpu.VMEM((ROWS_PER_W, EMBED_DIM), jnp.float32),
            pltpu.SemaphoreType.DMA,
            pltpu.SemaphoreType.DMA,
            pltpu.SemaphoreType.DMA,
        ],
    )(_body)
    return k(x3, table)


# speedup vs baseline: 1.0378x; 1.0004x over previous
import functools

import jax
import jax.numpy as jnp
from jax import lax
from jax.experimental import pallas as pl
from jax.experimental.pallas import tpu as pltpu
from jax.experimental.pallas import tpu_sc as plsc

BATCH = 4096
SEQ = 200
EMBED_DIM = 128
LANES = 16
NCHUNK = EMBED_DIM // LANES

NUM_CORES = 2
NUM_SUBCORES = 16
NW = NUM_CORES * NUM_SUBCORES
ROWS_PER_W = BATCH // NW
IDX_SPLIT = 2
IDX_CHUNK = SEQ // IDX_SPLIT


NBUF = 3


def _body(x_hbm, table_hbm, out_hbm, idx_v, buf_v, out_v, sem0, sem1, sem2):
    wid = lax.axis_index("s") * NUM_CORES + lax.axis_index("c")
    base = wid * ROWS_PER_W
    sems = (sem0, sem1, sem2)

    pltpu.sync_copy(x_hbm.at[pl.ds(base, ROWS_PER_W)], idx_v)

    inv_n = jnp.float32(1.0 / SEQ)

    def gather_row(r, slot):
        for c in range(IDX_SPLIT):
            pltpu.async_copy(
                table_hbm.at[idx_v.at[r, c]],
                buf_v.at[slot, pl.ds(c * IDX_CHUNK, IDX_CHUNK)],
                sems[slot],
            )

    def wait_row(slot):
        pltpu.make_async_copy(
            table_hbm.at[pl.ds(0, SEQ)], buf_v.at[slot], sems[slot]
        ).wait()

    def accum_row(r, slot):
        def accum(j, carry):
            return tuple(
                carry[c * NCHUNK + d]
                + buf_v[slot, c * IDX_CHUNK + j, pl.ds(d * LANES, LANES)]
                for c in range(IDX_SPLIT)
                for d in range(NCHUNK)
            )

        init = tuple(
            jnp.zeros((LANES,), jnp.float32) for _ in range(IDX_SPLIT * NCHUNK)
        )
        acc = lax.fori_loop(0, IDX_CHUNK, accum, init, unroll=4)
        for d in range(NCHUNK):
            out_v[r, pl.ds(d * LANES, LANES)] = (acc[d] + acc[NCHUNK + d]) * inv_n

    gather_row(0, 0)
    gather_row(1, 1)

    main_rows = ROWS_PER_W - (ROWS_PER_W % NBUF)

    @pl.loop(0, main_rows, step=NBUF)
    def ring(r0):
        for b in range(NBUF):
            r = r0 + b

            @pl.when(r + 2 < ROWS_PER_W)
            def _prefetch():
                gather_row(r + 2, (b + 2) % NBUF)

            wait_row(b)
            accum_row(r, b)

    for r in range(main_rows, ROWS_PER_W):
        wait_row(r % NBUF)
        accum_row(r, r % NBUF)

    pltpu.sync_copy(out_v, out_hbm.at[pl.ds(base, ROWS_PER_W)])


@jax.jit
def kernel(x, table):
    x3 = x.astype(jnp.int32).reshape(BATCH, IDX_SPLIT, IDX_CHUNK)
    mesh = plsc.VectorSubcoreMesh(core_axis_name="c", subcore_axis_name="s")
    k = functools.partial(
        pl.kernel,
        out_type=jax.ShapeDtypeStruct((BATCH, EMBED_DIM), jnp.float32),
        mesh=mesh,
        scratch_types=[
            pltpu.VMEM((ROWS_PER_W, IDX_SPLIT, IDX_CHUNK), jnp.int32),
            pltpu.VMEM((NBUF, SEQ, EMBED_DIM), jnp.float32),
            pltpu.VMEM((ROWS_PER_W, EMBED_DIM), jnp.float32),
            pltpu.SemaphoreType.DMA,
            pltpu.SemaphoreType.DMA,
            pltpu.SemaphoreType.DMA,
        ],
    )(_body)
    return k(x3, table)


# speedup vs baseline: 1.0408x; 1.0029x over previous
import functools

import jax
import jax.numpy as jnp
from jax import lax
from jax.experimental import pallas as pl
from jax.experimental.pallas import tpu as pltpu
from jax.experimental.pallas import tpu_sc as plsc

BATCH = 4096
SEQ = 200
EMBED_DIM = 128
LANES = 16
NCHUNK = EMBED_DIM // LANES

NUM_CORES = 2
NUM_SUBCORES = 16
NW = NUM_CORES * NUM_SUBCORES
ROWS_PER_W = BATCH // NW
IDX_SPLIT = 2
IDX_CHUNK = SEQ // IDX_SPLIT


NBUF = 3


def _body(x_hbm, table_hbm, out_hbm, idx_v, buf_v, out_v, sem0, sem1, sem2):
    wid = lax.axis_index("s") * NUM_CORES + lax.axis_index("c")
    base = wid * ROWS_PER_W
    sems = (sem0, sem1, sem2)

    pltpu.sync_copy(x_hbm.at[pl.ds(base, ROWS_PER_W)], idx_v)

    inv_n = jnp.float32(1.0 / SEQ)

    def gather_row(r, slot):
        for c in range(IDX_SPLIT):
            pltpu.async_copy(
                table_hbm.at[idx_v.at[r, c]],
                buf_v.at[slot, pl.ds(c * IDX_CHUNK, IDX_CHUNK)],
                sems[slot],
            )

    def wait_row(slot):
        pltpu.make_async_copy(
            table_hbm.at[pl.ds(0, SEQ)], buf_v.at[slot], sems[slot]
        ).wait()

    def accum_row(r, slot):
        def accum(j, carry):
            return tuple(
                carry[c * NCHUNK + d]
                + buf_v[slot, c * IDX_CHUNK + j, pl.ds(d * LANES, LANES)]
                for c in range(IDX_SPLIT)
                for d in range(NCHUNK)
            )

        init = tuple(
            jnp.zeros((LANES,), jnp.float32) for _ in range(IDX_SPLIT * NCHUNK)
        )
        acc = lax.fori_loop(0, IDX_CHUNK, accum, init, unroll=2)
        for d in range(NCHUNK):
            out_v[r, pl.ds(d * LANES, LANES)] = (acc[d] + acc[NCHUNK + d]) * inv_n

    gather_row(0, 0)
    gather_row(1, 1)

    main_rows = ROWS_PER_W - (ROWS_PER_W % NBUF)

    @pl.loop(0, main_rows, step=NBUF)
    def ring(r0):
        for b in range(NBUF):
            r = r0 + b

            @pl.when(r + 2 < ROWS_PER_W)
            def _prefetch():
                gather_row(r + 2, (b + 2) % NBUF)

            wait_row(b)
            accum_row(r, b)

    for r in range(main_rows, ROWS_PER_W):
        wait_row(r % NBUF)
        accum_row(r, r % NBUF)

    pltpu.sync_copy(out_v, out_hbm.at[pl.ds(base, ROWS_PER_W)])


@jax.jit
def kernel(x, table):
    x3 = x.astype(jnp.int32).reshape(BATCH, IDX_SPLIT, IDX_CHUNK)
    mesh = plsc.VectorSubcoreMesh(core_axis_name="c", subcore_axis_name="s")
    k = functools.partial(
        pl.kernel,
        out_type=jax.ShapeDtypeStruct((BATCH, EMBED_DIM), jnp.float32),
        mesh=mesh,
        scratch_types=[
            pltpu.VMEM((ROWS_PER_W, IDX_SPLIT, IDX_CHUNK), jnp.int32),
            pltpu.VMEM((NBUF, SEQ, EMBED_DIM), jnp.float32),
            pltpu.VMEM((ROWS_PER_W, EMBED_DIM), jnp.float32),
            pltpu.SemaphoreType.DMA,
            pltpu.SemaphoreType.DMA,
            pltpu.SemaphoreType.DMA,
        ],
    )(_body)
    return k(x3, table)


# speedup vs baseline: 1.0417x; 1.0009x over previous
import functools

import jax
import jax.numpy as jnp
from jax import lax
from jax.experimental import pallas as pl
from jax.experimental.pallas import tpu as pltpu
from jax.experimental.pallas import tpu_sc as plsc

BATCH = 4096
SEQ = 200
EMBED_DIM = 128
LANES = 16
NCHUNK = EMBED_DIM // LANES

NUM_CORES = 2
NUM_SUBCORES = 16
NW = NUM_CORES * NUM_SUBCORES
ROWS_PER_W = BATCH // NW
IDX_SPLIT = 2
IDX_CHUNK = SEQ // IDX_SPLIT


NBUF = 3


def _body(x_hbm, table_hbm, out_hbm, idx_v, buf_v, out_v, sem0, sem1, sem2):
    wid = lax.axis_index("s") * NUM_CORES + lax.axis_index("c")
    base = wid * ROWS_PER_W
    sems = (sem0, sem1, sem2)

    pltpu.sync_copy(x_hbm.at[pl.ds(base, ROWS_PER_W)], idx_v)

    inv_n = jnp.float32(1.0 / SEQ)

    def gather_row(r, slot):
        for c in range(IDX_SPLIT):
            pltpu.async_copy(
                table_hbm.at[idx_v.at[r, c]],
                buf_v.at[slot, pl.ds(c * IDX_CHUNK, IDX_CHUNK)],
                sems[slot],
            )

    def wait_row(slot):
        pltpu.make_async_copy(
            table_hbm.at[pl.ds(0, SEQ)], buf_v.at[slot], sems[slot]
        ).wait()

    def accum_row(r, slot):
        def accum(j, carry):
            return tuple(
                carry[c * NCHUNK + d]
                + buf_v[slot, c * IDX_CHUNK + j, pl.ds(d * LANES, LANES)]
                for c in range(IDX_SPLIT)
                for d in range(NCHUNK)
            )

        init = tuple(
            jnp.zeros((LANES,), jnp.float32) for _ in range(IDX_SPLIT * NCHUNK)
        )
        acc = lax.fori_loop(0, IDX_CHUNK, accum, init, unroll=1)
        for d in range(NCHUNK):
            out_v[r, pl.ds(d * LANES, LANES)] = (acc[d] + acc[NCHUNK + d]) * inv_n

    gather_row(0, 0)
    gather_row(1, 1)

    main_rows = ROWS_PER_W - (ROWS_PER_W % NBUF)

    @pl.loop(0, main_rows, step=NBUF)
    def ring(r0):
        for b in range(NBUF):
            r = r0 + b

            @pl.when(r + 2 < ROWS_PER_W)
            def _prefetch():
                gather_row(r + 2, (b + 2) % NBUF)

            wait_row(b)
            accum_row(r, b)

    for r in range(main_rows, ROWS_PER_W):
        wait_row(r % NBUF)
        accum_row(r, r % NBUF)

    pltpu.sync_copy(out_v, out_hbm.at[pl.ds(base, ROWS_PER_W)])


@jax.jit
def kernel(x, table):
    x3 = x.astype(jnp.int32).reshape(BATCH, IDX_SPLIT, IDX_CHUNK)
    mesh = plsc.VectorSubcoreMesh(core_axis_name="c", subcore_axis_name="s")
    k = functools.partial(
        pl.kernel,
        out_type=jax.ShapeDtypeStruct((BATCH, EMBED_DIM), jnp.float32),
        mesh=mesh,
        scratch_types=[
            pltpu.VMEM((ROWS_PER_W, IDX_SPLIT, IDX_CHUNK), jnp.int32),
            pltpu.VMEM((NBUF, SEQ, EMBED_DIM), jnp.float32),
            pltpu.VMEM((ROWS_PER_W, EMBED_DIM), jnp.float32),
            pltpu.SemaphoreType.DMA,
            pltpu.SemaphoreType.DMA,
            pltpu.SemaphoreType.DMA,
        ],
    )(_body)
    return k(x3, table)


# speedup vs baseline: 1.0498x; 1.0078x over previous
import functools

import jax
import jax.numpy as jnp
from jax import lax
from jax.experimental import pallas as pl
from jax.experimental.pallas import tpu as pltpu
from jax.experimental.pallas import tpu_sc as plsc

BATCH = 4096
SEQ = 200
EMBED_DIM = 128
LANES = 16
NCHUNK = EMBED_DIM // LANES

NUM_CORES = 2
NUM_SUBCORES = 16
NW = NUM_CORES * NUM_SUBCORES
ROWS_PER_W = BATCH // NW
IDX_SPLIT = 2
IDX_CHUNK = SEQ // IDX_SPLIT


NBUF = 3


def _body(x_hbm, table_hbm, out_hbm, idx_v, buf_v, out_v, sem0, sem1, sem2, semi):
    wid = lax.axis_index("s") * NUM_CORES + lax.axis_index("c")
    base = wid * ROWS_PER_W
    sems = (sem0, sem1, sem2)

    pltpu.sync_copy(x_hbm.at[pl.ds(base, 2)], idx_v.at[pl.ds(0, 2)])
    rest = pltpu.async_copy(
        x_hbm.at[pl.ds(base + 2, ROWS_PER_W - 2)],
        idx_v.at[pl.ds(2, ROWS_PER_W - 2)],
        semi,
    )

    inv_n = jnp.float32(1.0 / SEQ)

    def gather_row(r, slot):
        for c in range(IDX_SPLIT):
            pltpu.async_copy(
                table_hbm.at[idx_v.at[r, c]],
                buf_v.at[slot, pl.ds(c * IDX_CHUNK, IDX_CHUNK)],
                sems[slot],
            )

    def wait_row(slot):
        pltpu.make_async_copy(
            table_hbm.at[pl.ds(0, SEQ)], buf_v.at[slot], sems[slot]
        ).wait()

    def accum_row(r, slot):
        def accum(j, carry):
            return tuple(
                carry[c * NCHUNK + d]
                + buf_v[slot, c * IDX_CHUNK + j, pl.ds(d * LANES, LANES)]
                for c in range(IDX_SPLIT)
                for d in range(NCHUNK)
            )

        init = tuple(
            jnp.zeros((LANES,), jnp.float32) for _ in range(IDX_SPLIT * NCHUNK)
        )
        acc = lax.fori_loop(0, IDX_CHUNK, accum, init, unroll=1)
        for d in range(NCHUNK):
            out_v[r, pl.ds(d * LANES, LANES)] = (acc[d] + acc[NCHUNK + d]) * inv_n

    gather_row(0, 0)
    gather_row(1, 1)
    rest.wait()

    main_rows = ROWS_PER_W - (ROWS_PER_W % NBUF)

    @pl.loop(0, main_rows, step=NBUF)
    def ring(r0):
        for b in range(NBUF):
            r = r0 + b

            @pl.when(r + 2 < ROWS_PER_W)
            def _prefetch():
                gather_row(r + 2, (b + 2) % NBUF)

            wait_row(b)
            accum_row(r, b)

    for r in range(main_rows, ROWS_PER_W):
        wait_row(r % NBUF)
        accum_row(r, r % NBUF)

    pltpu.sync_copy(out_v, out_hbm.at[pl.ds(base, ROWS_PER_W)])


@jax.jit
def kernel(x, table):
    x3 = x.astype(jnp.int32).reshape(BATCH, IDX_SPLIT, IDX_CHUNK)
    mesh = plsc.VectorSubcoreMesh(core_axis_name="c", subcore_axis_name="s")
    k = functools.partial(
        pl.kernel,
        out_type=jax.ShapeDtypeStruct((BATCH, EMBED_DIM), jnp.float32),
        mesh=mesh,
        scratch_types=[
            pltpu.VMEM((ROWS_PER_W, IDX_SPLIT, IDX_CHUNK), jnp.int32),
            pltpu.VMEM((NBUF, SEQ, EMBED_DIM), jnp.float32),
            pltpu.VMEM((ROWS_PER_W, EMBED_DIM), jnp.float32),
            pltpu.SemaphoreType.DMA,
            pltpu.SemaphoreType.DMA,
            pltpu.SemaphoreType.DMA,
            pltpu.SemaphoreType.DMA,
        ],
    )(_body)
    return k(x3, table)
